# baseline (speedup 1.0000x reference)
import functools

import jax
import jax.numpy as jnp
from jax.experimental import pallas as pl
from jax.experimental.pallas import tpu as pltpu

NUM_EXPERTS = 64
EXPERT_CAPACITY = 160
BT = 1024


def _router_kernel(hsa_ref, hsb_ref, wa_ref, wb_ref, idx_ref, pmax_ref,
                   logits_ref, counts_ref):
    t = pl.program_id(1)

    @pl.when(t == 0)
    def _reset():
        counts_ref[...] = jnp.zeros_like(counts_ref)

    dn = (((1,), (1,)), ((), ()))
    logits = jax.lax.dot_general(
        hsa_ref[0], wa_ref[...], dn, preferred_element_type=jnp.float32)
    logits = logits + jax.lax.dot_general(
        hsb_ref[0], wb_ref[...], dn, preferred_element_type=jnp.float32)

    m = jnp.max(logits, axis=-1, keepdims=True)
    z = jnp.sum(jnp.exp(logits - m), axis=-1, keepdims=True)

    iota = jax.lax.broadcasted_iota(jnp.int32, logits.shape, 1)
    cand = jnp.where(logits == m, iota, NUM_EXPERTS)
    amin = jnp.min(cand, axis=-1, keepdims=True)
    oh = (iota == amin).astype(jnp.float32)

    row = jax.lax.broadcasted_iota(jnp.int32, (BT, BT), 0)
    col = jax.lax.broadcasted_iota(jnp.int32, (BT, BT), 1)
    tri = (row >= col).astype(jnp.float32)
    prio = jnp.dot(tri, oh, preferred_element_type=jnp.float32)
    prio = prio + counts_ref[...]
    counts_ref[...] = prio[BT - 1:BT, :]

    keep = prio <= float(EXPERT_CAPACITY)
    idx_ref[0] = jnp.where(keep, oh, 0.0).astype(jnp.int32)
    pmax_ref[0] = 1.0 / z
    logits_ref[0] = logits


@jax.jit
def kernel(hidden_states, W):
    G, T, H = hidden_states.shape
    E = W.shape[0]
    grid = (G, T // BT)
    out = pl.pallas_call(
        _router_kernel,
        grid=grid,
        in_specs=[
            pl.BlockSpec((1, BT, H // 2), lambda g, t: (g, t, 0)),
            pl.BlockSpec((1, BT, H // 2), lambda g, t: (g, t, 1)),
            pl.BlockSpec((E, H // 2), lambda g, t: (0, 0)),
            pl.BlockSpec((E, H // 2), lambda g, t: (0, 1)),
        ],
        out_specs=[
            pl.BlockSpec((1, BT, E), lambda g, t: (g, t, 0)),
            pl.BlockSpec((1, BT, 1), lambda g, t: (g, t, 0)),
            pl.BlockSpec((1, BT, E), lambda g, t: (g, t, 0)),
        ],
        out_shape=[
            jax.ShapeDtypeStruct((G, T, E), jnp.int32),
            jax.ShapeDtypeStruct((G, T, 1), jnp.float32),
            jax.ShapeDtypeStruct((G, T, E), jnp.float32),
        ],
        scratch_shapes=[pltpu.VMEM((1, E), jnp.float32)],
        compiler_params=pltpu.CompilerParams(
            dimension_semantics=("parallel", "arbitrary"),
            vmem_limit_bytes=120 * 1024 * 1024),
    )(hidden_states, hidden_states, W, W)
    expert_index, router_probs_max, router_logits = out
    return expert_index, router_probs_max, router_logits


# speedup vs baseline: 1.0249x; 1.0249x over previous
import functools

import jax
import jax.numpy as jnp
from jax.experimental import pallas as pl
from jax.experimental.pallas import tpu as pltpu

NUM_EXPERTS = 64
EXPERT_CAPACITY = 160
BT = 1024


def _router_kernel(hsa_ref, hsb_ref, wa_ref, wb_ref, idx_ref, pmax_ref,
                   logits_ref, counts_ref):
    t = pl.program_id(1)

    @pl.when(t == 0)
    def _reset():
        counts_ref[...] = jnp.zeros_like(counts_ref)

    dn = (((1,), (1,)), ((), ()))
    logits = jax.lax.dot_general(
        hsa_ref[0], wa_ref[...], dn, preferred_element_type=jnp.float32)
    logits = logits + jax.lax.dot_general(
        hsb_ref[0], wb_ref[...], dn, preferred_element_type=jnp.float32)

    m = jnp.max(logits, axis=-1, keepdims=True)
    idx_ref[0] = jnp.zeros(idx_ref.shape[1:], jnp.int32)
    pmax_ref[0] = m
    logits_ref[0] = logits


@jax.jit
def kernel(hidden_states, W):
    G, T, H = hidden_states.shape
    E = W.shape[0]
    grid = (G, T // BT)
    out = pl.pallas_call(
        _router_kernel,
        grid=grid,
        in_specs=[
            pl.BlockSpec((1, BT, H // 2), lambda g, t: (g, t, 0)),
            pl.BlockSpec((1, BT, H // 2), lambda g, t: (g, t, 1)),
            pl.BlockSpec((E, H // 2), lambda g, t: (0, 0)),
            pl.BlockSpec((E, H // 2), lambda g, t: (0, 1)),
        ],
        out_specs=[
            pl.BlockSpec((1, BT, E), lambda g, t: (g, t, 0)),
            pl.BlockSpec((1, BT, 1), lambda g, t: (g, t, 0)),
            pl.BlockSpec((1, BT, E), lambda g, t: (g, t, 0)),
        ],
        out_shape=[
            jax.ShapeDtypeStruct((G, T, E), jnp.int32),
            jax.ShapeDtypeStruct((G, T, 1), jnp.float32),
            jax.ShapeDtypeStruct((G, T, E), jnp.float32),
        ],
        scratch_shapes=[pltpu.VMEM((1, E), jnp.float32)],
        compiler_params=pltpu.CompilerParams(
            dimension_semantics=("parallel", "arbitrary"),
            vmem_limit_bytes=120 * 1024 * 1024),
    )(hidden_states, hidden_states, W, W)
    expert_index, router_probs_max, router_logits = out
    return expert_index, router_probs_max, router_logits


# speedup vs baseline: 1.2813x; 1.2502x over previous
import jax
import jax.numpy as jnp
from jax.experimental import pallas as pl
from jax.experimental.pallas import tpu as pltpu

NUM_EXPERTS = 64
EXPERT_CAPACITY = 160
BT = 1024


def _router_kernel(hs_ref, w_ref, idx_ref, pmax_ref, logits_ref, counts_ref):
    t = pl.program_id(1)

    @pl.when(t == 0)
    def _reset():
        counts_ref[...] = jnp.zeros_like(counts_ref)

    x = hs_ref[0]
    logits_t = jax.lax.dot_general(
        w_ref[...], x, (((1,), (1,)), ((), ())),
        preferred_element_type=jnp.float32)

    m = jnp.max(logits_t, axis=0, keepdims=True)
    z = jnp.sum(jnp.exp(logits_t - m), axis=0, keepdims=True)

    iota = jax.lax.broadcasted_iota(jnp.int32, logits_t.shape, 0)
    cand = jnp.where(logits_t == m, iota, NUM_EXPERTS)
    amin = jnp.min(cand, axis=0, keepdims=True)
    oh = (iota == amin).astype(jnp.float32)

    row = jax.lax.broadcasted_iota(jnp.int32, (BT, BT), 0)
    col = jax.lax.broadcasted_iota(jnp.int32, (BT, BT), 1)
    tri = (row <= col).astype(jnp.float32)
    prio = jnp.dot(oh, tri, preferred_element_type=jnp.float32)
    prio = prio + counts_ref[...]
    counts_ref[...] = prio[:, BT - 1:BT]

    keep = prio <= float(EXPERT_CAPACITY)
    idx_ref[0] = jnp.where(keep, oh, 0.0).astype(jnp.int32)
    pmax_ref[0] = 1.0 / z
    logits_ref[0] = logits_t


@jax.jit
def kernel(hidden_states, W):
    G, T, H = hidden_states.shape
    E = W.shape[0]
    grid = (G, T // BT)
    idx_t, pmax, logits_t = pl.pallas_call(
        _router_kernel,
        grid=grid,
        in_specs=[
            pl.BlockSpec((1, BT, H), lambda g, t: (g, t, 0)),
            pl.BlockSpec((E, H), lambda g, t: (0, 0)),
        ],
        out_specs=[
            pl.BlockSpec((1, E, BT), lambda g, t: (g, 0, t)),
            pl.BlockSpec((1, 1, BT), lambda g, t: (g, 0, t)),
            pl.BlockSpec((1, E, BT), lambda g, t: (g, 0, t)),
        ],
        out_shape=[
            jax.ShapeDtypeStruct((G, E, T), jnp.int32),
            jax.ShapeDtypeStruct((G, 1, T), jnp.float32),
            jax.ShapeDtypeStruct((G, E, T), jnp.float32),
        ],
        scratch_shapes=[pltpu.VMEM((E, 1), jnp.float32)],
        compiler_params=pltpu.CompilerParams(
            dimension_semantics=("parallel", "arbitrary")),
    )(hidden_states, W)
    expert_index = jnp.transpose(idx_t, (0, 2, 1))
    router_probs_max = jnp.transpose(pmax, (0, 2, 1))
    router_logits = jnp.transpose(logits_t, (0, 2, 1))
    return expert_index, router_probs_max, router_logits
